# baseline (speedup 1.0000x reference)
import functools

import jax
import jax.numpy as jnp
from jax import lax
from jax.experimental import pallas as pl
from jax.experimental.pallas import tpu as pltpu
from jax.experimental.pallas import tpu_sc as plsc

_B, _T, _D, _H = 1024, 200, 512, 4
_TIME_OFFSET = _H - 1
_LN_EPS = 1e-5
_BB = 24

_NC, _NS, _L = 2, 16, 16
_NW = _NC * _NS
_TP = 256
_RPW = _TP // _NW


@functools.partial(
    pl.kernel,
    mesh=plsc.VectorSubcoreMesh(core_axis_name="c", subcore_axis_name="s"),
    out_type=jax.ShapeDtypeStruct((_TP, _D), jnp.float32),
    scratch_types=[
        pltpu.VMEM((_RPW,), jnp.int32),
        pltpu.VMEM((_RPW, _D), jnp.float32),
        pltpu.VMEM((_D,), jnp.float32),
        pltpu.SemaphoreType.DMA,
    ],
)
def _sc_gather_bias(table_hbm, idx_hbm, type_hbm, out_hbm,
                    idx_v, rows_v, type_v, sem):
    wid = lax.axis_index("s") * _NC + lax.axis_index("c")
    base = wid * _RPW
    pltpu.sync_copy(idx_hbm.at[pl.ds(base, _RPW)], idx_v)
    pltpu.sync_copy(type_hbm, type_v)
    pltpu.async_copy(table_hbm.at[idx_v], rows_v, sem).wait()
    for r in range(_RPW):
        for j in range(_D // _L):
            sl = pl.ds(j * _L, _L)
            rows_v[r, sl] = rows_v[r, sl] + type_v[sl]
    pltpu.sync_copy(rows_v, out_hbm.at[pl.ds(base, _RPW)])


def _ln_body(bias_ref, g_ref, b_ref, x_ref, o_ref):
    x = x_ref[...] + bias_ref[...][None, :, :]
    mean = jnp.mean(x, axis=-1, keepdims=True)
    xc = x - mean
    var = jnp.mean(xc * xc, axis=-1, keepdims=True)
    o_ref[...] = xc * jax.lax.rsqrt(var + _LN_EPS) * g_ref[...] + b_ref[...]


@jax.jit
def kernel(tokens, t_idx, time_table, type_image, ln_gamma, ln_beta):
    idx = jnp.clip(t_idx.astype(jnp.int32) + _TIME_OFFSET, 0, _H - 1)
    idx_pad = jnp.zeros((_TP,), jnp.int32).at[:_T].set(idx)
    bias = _sc_gather_bias(time_table, idx_pad, type_image.reshape(_D))

    g = ln_gamma.reshape(1, _D)
    b = ln_beta.reshape(1, _D)
    return pl.pallas_call(
        _ln_body,
        grid=(pl.cdiv(_B, _BB),),
        in_specs=[
            pl.BlockSpec((_T, _D), lambda i: (0, 0)),
            pl.BlockSpec((1, _D), lambda i: (0, 0)),
            pl.BlockSpec((1, _D), lambda i: (0, 0)),
            pl.BlockSpec((_BB, _T, _D), lambda i: (i, 0, 0)),
        ],
        out_specs=pl.BlockSpec((_BB, _T, _D), lambda i: (i, 0, 0)),
        out_shape=jax.ShapeDtypeStruct((_B, _T, _D), jnp.float32),
        compiler_params=pltpu.CompilerParams(
            dimension_semantics=("parallel",),
        ),
    )(bias, g, b, tokens)
